# Initial kernel scaffold; baseline (speedup 1.0000x reference)
import jax
import jax.numpy as jnp
from jax.experimental import pallas as pl


def kernel(hidden_states, gate_w, expert_w1, expert_w2, expert_w3, shared_w1, shared_w2, shared_w3, shared_gate_w):
    raise NotImplementedError("write your pallas kernel here")


# speedup vs baseline: 1.0826x; 1.0826x over previous
import functools

import jax
import jax.numpy as jnp
from jax.experimental import pallas as pl
from jax.experimental.pallas import tpu as pltpu

NUM_EXPERTS = 8
TOP_K = 2
HIDDEN = 2048
EXPERT_FF = 1408
SHARED_FF = 5632
SEQ = 2048

TILE = 128
NT = (TOP_K * SEQ + NUM_EXPERTS * (TILE - 1) + TILE - 1) // TILE
PAD = NT * TILE


def _up_proj_kernel(s_ref, x_ref, w1_ref, w3_ref, h_ref):
    i = pl.program_id(0)

    @pl.when(i < s_ref[NT])
    def _():
        xb = x_ref[...]
        w1 = w1_ref[0].astype(jnp.bfloat16)
        w3 = w3_ref[0].astype(jnp.bfloat16)
        a = jax.lax.dot_general(xb, w1, (((1,), (1,)), ((), ())),
                                preferred_element_type=jnp.float32)
        u = jax.lax.dot_general(xb, w3, (((1,), (1,)), ((), ())),
                                preferred_element_type=jnp.float32)
        h = (a * jax.nn.sigmoid(a)) * u
        h_ref[...] = h.astype(jnp.bfloat16)


def _down_proj_kernel(s_ref, h_ref, w2_ref, y_ref):
    i = pl.program_id(0)

    @pl.when(i < s_ref[NT])
    def _():
        hb = h_ref[...]
        w2 = w2_ref[0].astype(jnp.bfloat16)
        y = jax.lax.dot_general(hb, w2, (((1,), (1,)), ((), ())),
                                preferred_element_type=jnp.float32)
        y_ref[...] = y.astype(jnp.bfloat16)


def _shared_up_kernel(x_ref, w1_ref, w3_ref, h_ref):
    xb = x_ref[...]
    w1 = w1_ref[...].astype(jnp.bfloat16)
    w3 = w3_ref[...].astype(jnp.bfloat16)
    a = jax.lax.dot_general(xb, w1, (((1,), (1,)), ((), ())),
                            preferred_element_type=jnp.float32)
    u = jax.lax.dot_general(xb, w3, (((1,), (1,)), ((), ())),
                            preferred_element_type=jnp.float32)
    h = (a * jax.nn.sigmoid(a)) * u
    h_ref[...] = h.astype(jnp.bfloat16)


def _shared_down_kernel(h_ref, w2_ref, y_ref):
    kc = pl.program_id(1)
    hb = h_ref[...]
    w2 = w2_ref[...].astype(jnp.bfloat16)
    y = jax.lax.dot_general(hb, w2, (((1,), (1,)), ((), ())),
                            preferred_element_type=jnp.float32)

    @pl.when(kc == 0)
    def _():
        y_ref[...] = y

    @pl.when(kc > 0)
    def _():
        y_ref[...] += y


def kernel(hidden_states, gate_w, expert_w1, expert_w2, expert_w3,
           shared_w1, shared_w2, shared_w3, shared_gate_w):
    b, s_len, d = hidden_states.shape
    x = hidden_states.reshape(-1, d)
    n_tok = x.shape[0]

    logits = x @ gate_w.T
    probs = jax.nn.softmax(logits.astype(jnp.float32), axis=1)
    rw, sel = jax.lax.top_k(probs, TOP_K)

    e_flat = sel.reshape(-1).astype(jnp.int32)
    onehot = (e_flat[:, None] == jnp.arange(NUM_EXPERTS, dtype=jnp.int32)[None, :]
              ).astype(jnp.int32)
    csum = jnp.cumsum(onehot, axis=0)
    counts = csum[-1]
    rank = jnp.sum((csum - onehot) * onehot, axis=1)
    padded = ((counts + TILE - 1) // TILE) * TILE
    pend = jnp.cumsum(padded)
    pstart = pend - padded
    dest_flat = pstart[e_flat].astype(jnp.int32) + rank
    token_ids = jnp.arange(n_tok * TOP_K, dtype=jnp.int32) // TOP_K
    rows_buf = jnp.zeros(PAD, jnp.int32).at[dest_flat].set(token_ids)
    num_active = (pend[-1] // TILE).astype(jnp.int32)
    tile_expert = jnp.searchsorted(pend, jnp.arange(NT, dtype=jnp.int32) * TILE,
                                   side='right').astype(jnp.int32)
    tile_expert = jnp.minimum(tile_expert, NUM_EXPERTS - 1)
    scalars = jnp.concatenate([tile_expert, num_active[None]])

    x_bf = x.astype(jnp.bfloat16)
    x_buf = x_bf[rows_buf]

    h_buf = pl.pallas_call(
        _up_proj_kernel,
        grid_spec=pltpu.PrefetchScalarGridSpec(
            num_scalar_prefetch=1,
            grid=(NT,),
            in_specs=[
                pl.BlockSpec((TILE, HIDDEN), lambda i, s: (i, 0)),
                pl.BlockSpec((1, EXPERT_FF, HIDDEN), lambda i, s: (s[i], 0, 0)),
                pl.BlockSpec((1, EXPERT_FF, HIDDEN), lambda i, s: (s[i], 0, 0)),
            ],
            out_specs=pl.BlockSpec((TILE, EXPERT_FF), lambda i, s: (i, 0)),
        ),
        out_shape=jax.ShapeDtypeStruct((PAD, EXPERT_FF), jnp.bfloat16),
    )(scalars, x_buf, expert_w1, expert_w3)

    y_buf = pl.pallas_call(
        _down_proj_kernel,
        grid_spec=pltpu.PrefetchScalarGridSpec(
            num_scalar_prefetch=1,
            grid=(NT,),
            in_specs=[
                pl.BlockSpec((TILE, EXPERT_FF), lambda i, s: (i, 0)),
                pl.BlockSpec((1, HIDDEN, EXPERT_FF), lambda i, s: (s[i], 0, 0)),
            ],
            out_specs=pl.BlockSpec((TILE, HIDDEN), lambda i, s: (i, 0)),
        ),
        out_shape=jax.ShapeDtypeStruct((PAD, HIDDEN), jnp.bfloat16),
    )(scalars, h_buf, expert_w2)

    FC = 11
    MT = 4
    fdim = SHARED_FF // FC
    mdim = n_tok // MT
    h_s = pl.pallas_call(
        _shared_up_kernel,
        grid=(FC, MT),
        in_specs=[
            pl.BlockSpec((mdim, HIDDEN), lambda f, m: (m, 0)),
            pl.BlockSpec((fdim, HIDDEN), lambda f, m: (f, 0)),
            pl.BlockSpec((fdim, HIDDEN), lambda f, m: (f, 0)),
        ],
        out_specs=pl.BlockSpec((mdim, fdim), lambda f, m: (m, f)),
        out_shape=jax.ShapeDtypeStruct((n_tok, SHARED_FF), jnp.bfloat16),
    )(x_bf, shared_w1, shared_w3)

    MD = 2
    md = n_tok // MD
    y_s = pl.pallas_call(
        _shared_down_kernel,
        grid=(MD, FC),
        in_specs=[
            pl.BlockSpec((md, fdim), lambda m, k: (m, k)),
            pl.BlockSpec((HIDDEN, fdim), lambda m, k: (0, k)),
        ],
        out_specs=pl.BlockSpec((md, HIDDEN), lambda m, k: (m, 0)),
        out_shape=jax.ShapeDtypeStruct((n_tok, HIDDEN), jnp.float32),
    )(h_s, shared_w2)

    sig = jax.nn.sigmoid(x @ shared_gate_w.T)
    yg = y_buf[dest_flat.reshape(n_tok, TOP_K)].astype(jnp.float32)
    out = jnp.sum(rw[..., None] * yg, axis=1) + sig * y_s
    return out.reshape(b, s_len, d)


# speedup vs baseline: 1.4452x; 1.3349x over previous
import functools

import jax
from jax import lax
import jax.numpy as jnp
from jax.experimental import pallas as pl
from jax.experimental.pallas import tpu as pltpu
from jax.experimental.pallas import tpu_sc as plsc

NUM_EXPERTS = 8
TOP_K = 2
HIDDEN = 2048
EXPERT_FF = 1408
SHARED_FF = 5632
SEQ = 2048

TILE = 256
NT = (TOP_K * SEQ + NUM_EXPERTS * (TILE - 1) + TILE - 1) // TILE
PAD = NT * TILE


def _up_proj_kernel(s_ref, x_ref, w1_ref, w3_ref, h_ref):
    i = pl.program_id(0)

    @pl.when(i < s_ref[NT])
    def _():
        xb = x_ref[...].astype(jnp.bfloat16)
        w1 = w1_ref[0].astype(jnp.bfloat16)
        w3 = w3_ref[0].astype(jnp.bfloat16)
        a = jax.lax.dot_general(xb, w1, (((1,), (1,)), ((), ())),
                                preferred_element_type=jnp.float32)
        u = jax.lax.dot_general(xb, w3, (((1,), (1,)), ((), ())),
                                preferred_element_type=jnp.float32)
        h = (a * jax.nn.sigmoid(a)) * u
        h_ref[...] = h.astype(jnp.bfloat16)


def _down_proj_kernel(s_ref, h_ref, w2_ref, w_ref, y_ref):
    i = pl.program_id(0)

    @pl.when(i < s_ref[NT])
    def _():
        hb = h_ref[...]
        w2 = w2_ref[0].astype(jnp.bfloat16)
        y = jax.lax.dot_general(hb, w2, (((1,), (1,)), ((), ())),
                                preferred_element_type=jnp.float32)
        y_ref[...] = y * w_ref[0]

    @pl.when(i >= s_ref[NT])
    def _():
        y_ref[...] = jnp.zeros_like(y_ref)


def _shared_up_kernel(x_ref, w1_ref, w3_ref, h_ref):
    xb = x_ref[...].astype(jnp.bfloat16)
    w1 = w1_ref[...].astype(jnp.bfloat16)
    w3 = w3_ref[...].astype(jnp.bfloat16)
    a = jax.lax.dot_general(xb, w1, (((1,), (1,)), ((), ())),
                            preferred_element_type=jnp.float32)
    u = jax.lax.dot_general(xb, w3, (((1,), (1,)), ((), ())),
                            preferred_element_type=jnp.float32)
    h = (a * jax.nn.sigmoid(a)) * u
    h_ref[...] = h.astype(jnp.bfloat16)


def _shared_down_kernel(h_ref, w2_ref, sig_ref, y_ref):
    kc = pl.program_id(1)
    nk = pl.num_programs(1)
    hb = h_ref[...]
    w2 = w2_ref[...].astype(jnp.bfloat16)
    y = jax.lax.dot_general(hb, w2, (((1,), (1,)), ((), ())),
                            preferred_element_type=jnp.float32)

    @pl.when(kc == 0)
    def _():
        y_ref[...] = y

    @pl.when(kc > 0)
    def _():
        y_ref[...] += y

    @pl.when(kc == nk - 1)
    def _():
        y_ref[...] *= sig_ref[...]


def _dispatch_sc_kernel(x_hbm, rows_hbm, xb_hbm, idx_v, rows_v, sem):
    wid = lax.axis_index("s") * 2 + lax.axis_index("c")
    per_w = PAD // 32
    CH = 16

    def body(c, carry):
        base = wid * per_w + c * CH
        pltpu.sync_copy(rows_hbm.at[pl.ds(base, CH)], idx_v)
        pltpu.async_copy(x_hbm.at[idx_v], rows_v, sem).wait()
        pltpu.sync_copy(rows_v, xb_hbm.at[pl.ds(base, CH)])
        return carry

    lax.fori_loop(0, per_w // CH, body, 0)


def _combine_sc_kernel(y_hbm, ys_hbm, dest_hbm, out_hbm,
                       idx_v, rows_v, acc_v, sem):
    wid = lax.axis_index("s") * 2 + lax.axis_index("c")
    ntok_w = SEQ // 32
    CH = 8

    def chunk_body(c, carry):
        base = wid * ntok_w + c * CH
        pltpu.sync_copy(dest_hbm.at[pl.ds(base * TOP_K, CH * TOP_K)], idx_v)
        pltpu.async_copy(y_hbm.at[idx_v], rows_v, sem).wait()
        pltpu.sync_copy(ys_hbm.at[pl.ds(base, CH)], acc_v)

        def lane_body(j, carry2):
            off = pl.ds(j * 16, 16)
            for t in range(CH):
                acc_v[t, off] = (acc_v[t, off]
                                 + rows_v[2 * t, off] + rows_v[2 * t + 1, off])
            return carry2

        lax.fori_loop(0, HIDDEN // 16, lane_body, 0)
        pltpu.sync_copy(acc_v, out_hbm.at[pl.ds(base, CH)])
        return carry

    lax.fori_loop(0, ntok_w // CH, chunk_body, 0)


def _dispatch_gather(x, rows_buf):
    return pl.kernel(
        _dispatch_sc_kernel,
        out_type=jax.ShapeDtypeStruct((PAD, HIDDEN), jnp.float32),
        mesh=plsc.VectorSubcoreMesh(core_axis_name="c", subcore_axis_name="s"),
        scratch_types=[
            pltpu.VMEM((16,), jnp.int32),
            pltpu.VMEM((16, HIDDEN), jnp.float32),
            pltpu.SemaphoreType.DMA,
        ],
    )(x, rows_buf)


def _combine(y_buf, y_s, dest_flat, n_tok):
    return pl.kernel(
        _combine_sc_kernel,
        out_type=jax.ShapeDtypeStruct((n_tok, HIDDEN), jnp.float32),
        mesh=plsc.VectorSubcoreMesh(core_axis_name="c", subcore_axis_name="s"),
        scratch_types=[
            pltpu.VMEM((16,), jnp.int32),
            pltpu.VMEM((16, HIDDEN), jnp.float32),
            pltpu.VMEM((8, HIDDEN), jnp.float32),
            pltpu.SemaphoreType.DMA,
        ],
    )(y_buf, y_s, dest_flat)


def kernel(hidden_states, gate_w, expert_w1, expert_w2, expert_w3,
           shared_w1, shared_w2, shared_w3, shared_gate_w):
    b, s_len, d = hidden_states.shape
    x = hidden_states.reshape(-1, d)
    n_tok = x.shape[0]

    logits = x @ gate_w.T
    probs = jax.nn.softmax(logits.astype(jnp.float32), axis=1)
    rw, sel = jax.lax.top_k(probs, TOP_K)

    e_flat = sel.reshape(-1).astype(jnp.int32)
    onehot = (e_flat[:, None] == jnp.arange(NUM_EXPERTS, dtype=jnp.int32)[None, :]
              ).astype(jnp.int32)
    csum = jnp.cumsum(onehot, axis=0)
    counts = csum[-1]
    rank = jnp.sum((csum - onehot) * onehot, axis=1)
    padded = ((counts + TILE - 1) // TILE) * TILE
    pend = jnp.cumsum(padded)
    pstart = pend - padded
    dest_flat = pstart[e_flat].astype(jnp.int32) + rank
    token_ids = jnp.arange(n_tok * TOP_K, dtype=jnp.int32) // TOP_K
    pad_fill = jnp.arange(PAD, dtype=jnp.int32) % n_tok
    rows_buf = pad_fill.at[dest_flat].set(token_ids)
    w_flat = jnp.zeros(PAD, jnp.float32).at[dest_flat].set(
        rw.reshape(-1).astype(jnp.float32))
    w_buf = w_flat.reshape(NT, TILE, 1)
    num_active = (pend[-1] // TILE).astype(jnp.int32)
    tile_expert = jnp.searchsorted(pend, jnp.arange(NT, dtype=jnp.int32) * TILE,
                                   side='right').astype(jnp.int32)
    tile_expert = jnp.minimum(tile_expert, NUM_EXPERTS - 1)
    scalars = jnp.concatenate([tile_expert, num_active[None]])

    FC = 11
    MT = 4
    fdim = SHARED_FF // FC
    mdim = n_tok // MT
    h_s = pl.pallas_call(
        _shared_up_kernel,
        grid=(FC, MT),
        in_specs=[
            pl.BlockSpec((mdim, HIDDEN), lambda f, m: (m, 0)),
            pl.BlockSpec((fdim, HIDDEN), lambda f, m: (f, 0)),
            pl.BlockSpec((fdim, HIDDEN), lambda f, m: (f, 0)),
        ],
        out_specs=pl.BlockSpec((mdim, fdim), lambda f, m: (m, f)),
        out_shape=jax.ShapeDtypeStruct((n_tok, SHARED_FF), jnp.bfloat16),
    )(x, shared_w1, shared_w3)

    x_buf = _dispatch_gather(x, rows_buf)

    h_buf = pl.pallas_call(
        _up_proj_kernel,
        grid_spec=pltpu.PrefetchScalarGridSpec(
            num_scalar_prefetch=1,
            grid=(NT,),
            in_specs=[
                pl.BlockSpec((TILE, HIDDEN), lambda i, s: (i, 0)),
                pl.BlockSpec((1, EXPERT_FF, HIDDEN), lambda i, s: (s[i], 0, 0)),
                pl.BlockSpec((1, EXPERT_FF, HIDDEN), lambda i, s: (s[i], 0, 0)),
            ],
            out_specs=pl.BlockSpec((TILE, EXPERT_FF), lambda i, s: (i, 0)),
        ),
        out_shape=jax.ShapeDtypeStruct((PAD, EXPERT_FF), jnp.bfloat16),
    )(scalars, x_buf, expert_w1, expert_w3)

    y_buf = pl.pallas_call(
        _down_proj_kernel,
        grid_spec=pltpu.PrefetchScalarGridSpec(
            num_scalar_prefetch=1,
            grid=(NT,),
            in_specs=[
                pl.BlockSpec((TILE, EXPERT_FF), lambda i, s: (i, 0)),
                pl.BlockSpec((1, HIDDEN, EXPERT_FF), lambda i, s: (s[i], 0, 0)),
                pl.BlockSpec((1, TILE, 1), lambda i, s: (i, 0, 0)),
            ],
            out_specs=pl.BlockSpec((TILE, HIDDEN), lambda i, s: (i, 0)),
        ),
        out_shape=jax.ShapeDtypeStruct((PAD, HIDDEN), jnp.float32),
    )(scalars, h_buf, expert_w2, w_buf)

    sig = jax.nn.sigmoid((x @ shared_gate_w.T).astype(jnp.float32))
    MD = 2
    md = n_tok // MD
    y_s = pl.pallas_call(
        _shared_down_kernel,
        grid=(MD, FC),
        in_specs=[
            pl.BlockSpec((md, fdim), lambda m, k: (m, k)),
            pl.BlockSpec((HIDDEN, fdim), lambda m, k: (0, k)),
            pl.BlockSpec((md, 1), lambda m, k: (m, 0)),
        ],
        out_specs=pl.BlockSpec((md, HIDDEN), lambda m, k: (m, 0)),
        out_shape=jax.ShapeDtypeStruct((n_tok, HIDDEN), jnp.float32),
    )(h_s, shared_w2, sig)

    out = _combine(y_buf, y_s, dest_flat, n_tok)
    return out.reshape(b, s_len, d)
